# baseline (speedup 1.0000x reference)
import jax
import jax.numpy as jnp
from jax import lax
from jax.experimental import pallas as pl
from jax.experimental.pallas import tpu as pltpu
from jax.experimental.pallas import tpu_sc as plsc
B = 16
N = 200000
H = 720
W = 1280
NC = 2
NS = 16
L = 16
BAND = H // NS
CHN = 2048
NFULL = N // CHN
TAILC = 1280
TAILE = N - NFULL * CHN - TAILC
GUNROLL = 8


def _sc_body(x_hbm, xtail_hbm, out_hbm, band, ev0, ev1, tbuf, ob0, ob1, ob2,
             se0, se1, so0, so1):
    c = lax.axis_index("c")
    s = lax.axis_index("s")
    row_lo = s * BAND

    lane = lax.iota(jnp.int32, L)
    lane2 = lane * 2
    neg1 = jnp.full((L,), -1, jnp.int32)
    falses = jnp.zeros((L,), jnp.bool_)

    def _init(i, _):
        base = i * (L * 8)
        for u in range(8):
            band[pl.ds(base + u * L, L)] = neg1
        return 0
    lax.fori_loop(0, BAND * W // (L * 8), _init, 0)

    uband = jnp.uint32(BAND)

    def _start(b, ci, buf, sem, n=CHN):
        dst = buf if n == CHN else buf.at[:, pl.ds(0, n)]
        pltpu.async_copy(x_hbm.at[b, :, pl.ds(ci * CHN, n)], dst, sem)

    def _wait(b, ci, buf, sem, n=CHN):
        dst = buf if n == CHN else buf.at[:, pl.ds(0, n)]
        pltpu.make_async_copy(
            x_hbm.at[b, :, pl.ds(ci * CHN, n)], dst, sem).wait()

    def _tail_events(b):
        pltpu.sync_copy(xtail_hbm.at[b], tbuf)
        lane4 = lane * 4
        for g in range(TAILE // L):
            idx = lane4 + (g * (L * 4) + 1)
            xv = plsc.load_gather(tbuf, [idx])
            yv = plsc.load_gather(tbuf, [idx + 1])
            pv = plsc.load_gather(tbuf, [idx + 2])
            rl = yv.astype(jnp.int32) - row_lo
            m = lax.bitcast_convert_type(rl, jnp.uint32) < uband
            pix = rl * W + xv.astype(jnp.int32)
            _, last = plsc.scan_count(pix, mask=m)
            plsc.store_scatter(band, [pix], pv.astype(jnp.int32),
                               mask=m & last)

    def _ev_group(buf, g):
        base = g * L
        xv = buf[1, pl.ds(base, L)]
        yv = buf[2, pl.ds(base, L)]
        pv = buf[3, pl.ds(base, L)]
        rl = yv.astype(jnp.int32) - row_lo
        m = lax.bitcast_convert_type(rl, jnp.uint32) < uband
        pix = rl * W + xv.astype(jnp.int32)
        return m, pix, pv.astype(jnp.int32)

    def _process(buf, ci, ngroups):
        del ci

        def _g(j, _):
            for u in range(GUNROLL):
                g = j * GUNROLL + u
                m, pix, pol = _ev_group(buf, g)
                _, last = plsc.scan_count(pix, mask=m)
                plsc.store_scatter(band, [pix], pol, mask=m & last)
            return 0
        lax.fori_loop(0, ngroups // GUNROLL, _g, 0)

    def _batch(bi, _):
        b = c * 8 + bi

        _start(b, 0, ev0, se0)
        _start(b, 1, ev1, se1)

        def _pair(i, _):
            ci0 = i * 2
            _wait(b, ci0, ev0, se0)
            _process(ev0, ci0, CHN // L)
            _start(b, ci0 + 2, ev0, se0)
            _wait(b, ci0 + 1, ev1, se1)
            _process(ev1, ci0 + 1, CHN // L)

            @pl.when(i < NFULL // 2 - 1)
            def _():
                _start(b, ci0 + 3, ev1, se1)
            return 0
        lax.fori_loop(0, NFULL // 2, _pair, 0)

        _wait(b, NFULL - 1, ev0, se0)
        _process(ev0, NFULL - 1, CHN // L)
        _start(b, NFULL, ev1, se1, n=TAILC)
        _wait(b, NFULL, ev1, se1, n=TAILC)
        _process(ev1, NFULL, TAILC // L)
        _tail_events(b)

        osems = (so0, so1)

        def _row_slices(r):
            gr = row_lo + r
            tr = lax.shift_right_logical(gr, 3)
            sr = lax.bitwise_and(gr, 7)
            return [out_hbm.at[b, ch, tr, :, pl.ds(sr, 1), :]
                    for ch in range(3)]

        def _wait_row(r, p, sem):
            for ch, dst in enumerate(_row_slices(r)):
                pltpu.make_async_copy((ob0, ob1, ob2)[ch].at[p], dst,
                                      sem).wait()

        def _do_row(r, p, sem):
            rbase = r * W

            def _tc(tc, _):
                for u in range(8):
                    pbase = rbase + tc * 128 + u * L
                    key = band[pl.ds(pbase, L)]
                    band[pl.ds(pbase, L)] = neg1
                    touched = key >= 0
                    polb = (key & 1) == 1
                    y0 = jnp.where(touched & polb, 0.0, 255.0)
                    y0 = y0.astype(jnp.float32)
                    y1 = jnp.where(touched & (~polb), 0.0, 255.0)
                    y1 = y1.astype(jnp.float32)
                    ob0[p, tc, 0, pl.ds(u * L, L)] = y0
                    ob1[p, tc, 0, pl.ds(u * L, L)] = y1
                    ob2[p, tc, 0, pl.ds(u * L, L)] = y0 + y1
                return 0
            lax.fori_loop(0, 10, _tc, 0)
            for ch, dst in enumerate(_row_slices(r)):
                pltpu.async_copy((ob0, ob1, ob2)[ch].at[p], dst, sem)

        def _rowpair(i, _):
            @pl.when(i > 0)
            def _():
                _wait_row(2 * i - 2, 0, so0)
            _do_row(2 * i, 0, so0)

            @pl.when(i > 0)
            def _():
                _wait_row(2 * i - 1, 1, so1)
            _do_row(2 * i + 1, 1, so1)
            return 0
        lax.fori_loop(0, BAND // 2, _rowpair, 0)

        _wait_row(BAND - 3, 0, so0)
        _do_row(BAND - 1, 0, so0)
        _wait_row(BAND - 2, 1, so1)
        _wait_row(BAND - 1, 0, so0)
        return 0
    lax.fori_loop(0, B // NC, _batch, 0)


@jax.jit
def _e2img(xt, xtail):
    mesh = plsc.VectorSubcoreMesh(
        core_axis_name="c", subcore_axis_name="s", num_cores=NC, num_subcores=NS)
    f = pl.kernel(
        _sc_body,
        out_type=jax.ShapeDtypeStruct(
            (B, 3, H // 8, W // 128, 8, 128), jnp.float32),
        mesh=mesh,
        scratch_types=[
            pltpu.VMEM((BAND * W,), jnp.int32),
            pltpu.VMEM((4, CHN), jnp.float32),
            pltpu.VMEM((4, CHN), jnp.float32),
            pltpu.VMEM((TAILE * 4,), jnp.float32),
            pltpu.VMEM((2, W // 128, 1, 128), jnp.float32),
            pltpu.VMEM((2, W // 128, 1, 128), jnp.float32),
            pltpu.VMEM((2, W // 128, 1, 128), jnp.float32),
            pltpu.SemaphoreType.DMA,
            pltpu.SemaphoreType.DMA,
            pltpu.SemaphoreType.DMA,
            pltpu.SemaphoreType.DMA,
        ],
        compiler_params=pltpu.CompilerParams(needs_layout_passes=False),
    )
    return f(xt, xtail)


def kernel(x):
    xt = x.transpose(0, 2, 1)
    xtail = x[:, N - TAILE:, :].reshape(B, TAILE * 4)
    out = _e2img(xt, xtail)
    return out.transpose(0, 2, 4, 3, 5, 1).reshape(B, H, W, 3)


# speedup vs baseline: 1.3615x; 1.3615x over previous
import jax
import jax.numpy as jnp
from jax import lax
from jax.experimental import pallas as pl
from jax.experimental.pallas import tpu as pltpu
from jax.experimental.pallas import tpu_sc as plsc
B = 16
N = 200000
H = 720
W = 1280
NC = 2
NS = 16
L = 16
BAND = H // NS
CHN = 2048
NFULL = N // CHN
TAILC = 1280
TAILE = N - NFULL * CHN - TAILC
GUNROLL = 8


def _sc_body(x_hbm, xtail_hbm, out_hbm, band, ev0, ev1, tbuf, pixbuf, keybuf,
             ob0, ob1, ob2, se0, se1, so0, so1):
    c = lax.axis_index("c")
    s = lax.axis_index("s")
    row_lo = s * BAND

    lane = lax.iota(jnp.int32, L)
    lane2 = lane * 2
    neg1 = jnp.full((L,), -1, jnp.int32)
    falses = jnp.zeros((L,), jnp.bool_)

    def _init(i, _):
        base = i * (L * 8)
        for u in range(8):
            band[pl.ds(base + u * L, L)] = neg1
        return 0
    lax.fori_loop(0, BAND * W // (L * 8), _init, 0)

    uband = jnp.uint32(BAND)

    def _start(b, ci, buf, sem, n=CHN):
        dst = buf if n == CHN else buf.at[:, pl.ds(0, n)]
        pltpu.async_copy(x_hbm.at[b, :, pl.ds(ci * CHN, n)], dst, sem)

    def _wait(b, ci, buf, sem, n=CHN):
        dst = buf if n == CHN else buf.at[:, pl.ds(0, n)]
        pltpu.make_async_copy(
            x_hbm.at[b, :, pl.ds(ci * CHN, n)], dst, sem).wait()

    def _tail_events(b):
        pltpu.sync_copy(xtail_hbm.at[b], tbuf)
        lane4 = lane * 4
        for g in range(TAILE // L):
            idx = lane4 + (g * (L * 4) + 1)
            xv = plsc.load_gather(tbuf, [idx])
            yv = plsc.load_gather(tbuf, [idx + 1])
            pv = plsc.load_gather(tbuf, [idx + 2])
            rl = yv.astype(jnp.int32) - row_lo
            m = lax.bitcast_convert_type(rl, jnp.uint32) < uband
            pix = rl * W + xv.astype(jnp.int32)
            key = ((N - TAILE + g * L) * 2) + lane2 + pv.astype(jnp.int32)
            plsc.store_scatter(band, [pix], key, mask=m)
            back = plsc.load_gather(band, [pix], mask=m)
            need = m & (back < key)

            def _fix(nd, pix=pix, key=key):
                plsc.store_scatter(band, [pix], key, mask=nd)
                bk = plsc.load_gather(band, [pix], mask=nd)
                return nd & (bk < key)
            lax.while_loop(lambda nd: jnp.any(nd), _fix, need)

    upix = jnp.uint32(BAND * W)

    def _ev_group(buf, kvec0, g):
        base = g * L
        xv = buf[1, pl.ds(base, L)]
        yv = buf[2, pl.ds(base, L)]
        pv = buf[3, pl.ds(base, L)]
        rl = yv.astype(jnp.int32) - row_lo
        m = lax.bitcast_convert_type(rl, jnp.uint32) < uband
        pix = rl * W + xv.astype(jnp.int32)
        key = (kvec0 + g * (L * 2)) + pv.astype(jnp.int32)
        return m, pix, key

    def _process(buf, ci, ngroups):
        kvec0 = ci * (CHN * 2) + lane2

        def _p1(j, _):
            for u in range(GUNROLL):
                g = j * GUNROLL + u
                m, pix, key = _ev_group(buf, kvec0, g)
                plsc.store_scatter(band, [pix], key, mask=m)
                pixbuf[pl.ds(g * L, L)] = pix
                keybuf[pl.ds(g * L, L)] = key
            return 0
        lax.fori_loop(0, ngroups // GUNROLL, _p1, 0)

        def _p2(j, acc):
            for u in range(GUNROLL):
                g = j * GUNROLL + u
                pix = pixbuf[pl.ds(g * L, L)]
                key = keybuf[pl.ds(g * L, L)]
                m = lax.bitcast_convert_type(pix, jnp.uint32) < upix
                back = plsc.load_gather(band, [pix], mask=m)
                acc = acc | (m & (back < key))
            return acc
        acc = lax.fori_loop(0, ngroups // GUNROLL, _p2, falses)

        def _fixpass(_acc):
            def _fg(g, a):
                pix = pixbuf[pl.ds(g * L, L)]
                key = keybuf[pl.ds(g * L, L)]
                m = lax.bitcast_convert_type(pix, jnp.uint32) < upix
                back = plsc.load_gather(band, [pix], mask=m)
                n = m & (back < key)
                plsc.store_scatter(band, [pix], key, mask=n)
                back2 = plsc.load_gather(band, [pix], mask=n)
                return a | (n & (back2 < key))
            return lax.fori_loop(0, ngroups, _fg, falses)
        lax.while_loop(lambda a: jnp.any(a), _fixpass, acc)

    def _batch(bi, _):
        b = c * 8 + bi

        _start(b, 0, ev0, se0)
        _start(b, 1, ev1, se1)

        def _pair(i, _):
            ci0 = i * 2
            _wait(b, ci0, ev0, se0)
            _process(ev0, ci0, CHN // L)
            _start(b, ci0 + 2, ev0, se0)
            _wait(b, ci0 + 1, ev1, se1)
            _process(ev1, ci0 + 1, CHN // L)

            @pl.when(i < NFULL // 2 - 1)
            def _():
                _start(b, ci0 + 3, ev1, se1)
            return 0
        lax.fori_loop(0, NFULL // 2, _pair, 0)

        _wait(b, NFULL - 1, ev0, se0)
        _process(ev0, NFULL - 1, CHN // L)
        _start(b, NFULL, ev1, se1, n=TAILC)
        _wait(b, NFULL, ev1, se1, n=TAILC)
        _process(ev1, NFULL, TAILC // L)
        _tail_events(b)

        osems = (so0, so1)

        def _row_slices(r):
            gr = row_lo + r
            tr = lax.shift_right_logical(gr, 3)
            sr = lax.bitwise_and(gr, 7)
            return [out_hbm.at[b, ch, tr, :, pl.ds(sr, 1), :]
                    for ch in range(3)]

        def _wait_row(r, p, sem):
            for ch, dst in enumerate(_row_slices(r)):
                pltpu.make_async_copy((ob0, ob1, ob2)[ch].at[p], dst,
                                      sem).wait()

        def _do_row(r, p, sem):
            rbase = r * W

            def _tc(tc, _):
                for u in range(8):
                    pbase = rbase + tc * 128 + u * L
                    key = band[pl.ds(pbase, L)]
                    band[pl.ds(pbase, L)] = neg1
                    touched = key >= 0
                    polb = (key & 1) == 1
                    y0 = jnp.where(touched & polb, 0.0, 255.0)
                    y0 = y0.astype(jnp.float32)
                    y1 = jnp.where(touched & (~polb), 0.0, 255.0)
                    y1 = y1.astype(jnp.float32)
                    ob0[p, tc, 0, pl.ds(u * L, L)] = y0
                    ob1[p, tc, 0, pl.ds(u * L, L)] = y1
                    ob2[p, tc, 0, pl.ds(u * L, L)] = y0 + y1
                return 0
            lax.fori_loop(0, 10, _tc, 0)
            for ch, dst in enumerate(_row_slices(r)):
                pltpu.async_copy((ob0, ob1, ob2)[ch].at[p], dst, sem)

        def _rowpair(i, _):
            @pl.when(i > 0)
            def _():
                _wait_row(2 * i - 2, 0, so0)
            _do_row(2 * i, 0, so0)

            @pl.when(i > 0)
            def _():
                _wait_row(2 * i - 1, 1, so1)
            _do_row(2 * i + 1, 1, so1)
            return 0
        lax.fori_loop(0, BAND // 2, _rowpair, 0)

        _wait_row(BAND - 3, 0, so0)
        _do_row(BAND - 1, 0, so0)
        _wait_row(BAND - 2, 1, so1)
        _wait_row(BAND - 1, 0, so0)
        return 0
    lax.fori_loop(0, B // NC, _batch, 0)


@jax.jit
def _e2img(xt, xtail):
    mesh = plsc.VectorSubcoreMesh(
        core_axis_name="c", subcore_axis_name="s", num_cores=NC, num_subcores=NS)
    f = pl.kernel(
        _sc_body,
        out_type=jax.ShapeDtypeStruct(
            (B, 3, H // 8, W // 128, 8, 128), jnp.float32),
        mesh=mesh,
        scratch_types=[
            pltpu.VMEM((BAND * W,), jnp.int32),
            pltpu.VMEM((4, CHN), jnp.float32),
            pltpu.VMEM((4, CHN), jnp.float32),
            pltpu.VMEM((TAILE * 4,), jnp.float32),
            pltpu.VMEM((CHN,), jnp.int32),
            pltpu.VMEM((CHN,), jnp.int32),
            pltpu.VMEM((2, W // 128, 1, 128), jnp.float32),
            pltpu.VMEM((2, W // 128, 1, 128), jnp.float32),
            pltpu.VMEM((2, W // 128, 1, 128), jnp.float32),
            pltpu.SemaphoreType.DMA,
            pltpu.SemaphoreType.DMA,
            pltpu.SemaphoreType.DMA,
            pltpu.SemaphoreType.DMA,
        ],
        compiler_params=pltpu.CompilerParams(needs_layout_passes=False),
    )
    return f(xt, xtail)


def kernel(x):
    xt = x.transpose(0, 2, 1)
    xtail = x[:, N - TAILE:, :].reshape(B, TAILE * 4)
    out = _e2img(xt, xtail)
    return out.transpose(0, 2, 4, 3, 5, 1).reshape(B, H, W, 3)


# speedup vs baseline: 1.6579x; 1.2177x over previous
import jax
import jax.numpy as jnp
from jax import lax
from jax.experimental import pallas as pl
from jax.experimental.pallas import tpu as pltpu
from jax.experimental.pallas import tpu_sc as plsc
B = 16
N = 200000
H = 720
W = 1280
NC = 2
NS = 16
L = 16
BAND = H // NS
CHN = 2048
NFULL = N // CHN
TAILC = 1280
TAILE = N - NFULL * CHN - TAILC
GUNROLL = 4


def _sc_body(x_hbm, xtail_hbm, out_hbm, band, ev0, ev1, tbuf,
             ob0, ob1, ob2, se0, se1, so0, so1):
    c = lax.axis_index("c")
    s = lax.axis_index("s")
    row_lo = s * BAND

    lane = lax.iota(jnp.int32, L)
    lane2 = lane * 2
    neg1 = jnp.full((L,), -1, jnp.int32)
    falses = jnp.zeros((L,), jnp.bool_)

    def _init(i, _):
        base = i * (L * 8)
        for u in range(8):
            band[pl.ds(base + u * L, L)] = neg1
        return 0
    lax.fori_loop(0, BAND * W // (L * 8), _init, 0)

    uband = jnp.uint32(BAND)

    def _start(b, ci, buf, sem, n=CHN):
        dst = buf if n == CHN else buf.at[:, pl.ds(0, n)]
        pltpu.async_copy(x_hbm.at[b, :, pl.ds(ci * CHN, n)], dst, sem)

    def _wait(b, ci, buf, sem, n=CHN):
        dst = buf if n == CHN else buf.at[:, pl.ds(0, n)]
        pltpu.make_async_copy(
            x_hbm.at[b, :, pl.ds(ci * CHN, n)], dst, sem).wait()

    def _tail_events(b):
        pltpu.sync_copy(xtail_hbm.at[b], tbuf)
        lane4 = lane * 4
        for g in range(TAILE // L):
            idx = lane4 + (g * (L * 4) + 1)
            xv = plsc.load_gather(tbuf, [idx])
            yv = plsc.load_gather(tbuf, [idx + 1])
            pv = plsc.load_gather(tbuf, [idx + 2])
            rl = yv.astype(jnp.int32) - row_lo
            m = lax.bitcast_convert_type(rl, jnp.uint32) < uband
            pix = rl * W + xv.astype(jnp.int32)
            key = ((N - TAILE + g * L) * 2) + lane2 + pv.astype(jnp.int32)
            plsc.store_scatter(band, [pix], key, mask=m)
            back = plsc.load_gather(band, [pix], mask=m)
            need = m & (back < key)

            def _fix(nd, pix=pix, key=key):
                plsc.store_scatter(band, [pix], key, mask=nd)
                bk = plsc.load_gather(band, [pix], mask=nd)
                return nd & (bk < key)
            lax.while_loop(lambda nd: jnp.any(nd), _fix, need)

    upix = jnp.uint32(BAND * W)

    def _ev_group(buf, kvec0, g):
        base = g * L
        xv = buf[1, pl.ds(base, L)]
        yv = buf[2, pl.ds(base, L)]
        pv = buf[3, pl.ds(base, L)]
        rl = yv.astype(jnp.int32) - row_lo
        m = lax.bitcast_convert_type(rl, jnp.uint32) < uband
        pix = rl * W + xv.astype(jnp.int32)
        key = (kvec0 + g * (L * 2)) + pv.astype(jnp.int32)
        return m, pix, key

    def _process(buf, ci, ngroups):
        kvec0 = ci * (CHN * 2) + lane2

        def _g(j, acc):
            for u in range(GUNROLL):
                g = j * GUNROLL + u
                m, pix, key = _ev_group(buf, kvec0, g)
                plsc.store_scatter(band, [pix], key, mask=m)
                back = plsc.load_gather(band, [pix], mask=m)
                acc = acc | (m & (back < key))
            return acc
        acc = lax.fori_loop(0, ngroups // GUNROLL, _g, falses)

        def _fixpass(_acc):
            def _fg(g, a):
                m, pix, key = _ev_group(buf, kvec0, g)
                back = plsc.load_gather(band, [pix], mask=m)
                n = m & (back < key)
                plsc.store_scatter(band, [pix], key, mask=n)
                back2 = plsc.load_gather(band, [pix], mask=n)
                return a | (n & (back2 < key))
            return lax.fori_loop(0, ngroups, _fg, falses)
        lax.while_loop(lambda a: jnp.any(a), _fixpass, acc)

    def _batch(bi, _):
        b = c * 8 + bi

        _start(b, 0, ev0, se0)
        _start(b, 1, ev1, se1)

        def _pair(i, _):
            ci0 = i * 2
            _wait(b, ci0, ev0, se0)
            _process(ev0, ci0, CHN // L)
            _start(b, ci0 + 2, ev0, se0)
            _wait(b, ci0 + 1, ev1, se1)
            _process(ev1, ci0 + 1, CHN // L)

            @pl.when(i < NFULL // 2 - 1)
            def _():
                _start(b, ci0 + 3, ev1, se1)
            return 0
        lax.fori_loop(0, NFULL // 2, _pair, 0)

        _wait(b, NFULL - 1, ev0, se0)
        _process(ev0, NFULL - 1, CHN // L)
        _start(b, NFULL, ev1, se1, n=TAILC)
        _wait(b, NFULL, ev1, se1, n=TAILC)
        _process(ev1, NFULL, TAILC // L)
        _tail_events(b)

        osems = (so0, so1)

        def _row_slices(r):
            gr = row_lo + r
            tr = lax.shift_right_logical(gr, 3)
            sr = lax.bitwise_and(gr, 7)
            return [out_hbm.at[b, ch, tr, :, pl.ds(sr, 1), :]
                    for ch in range(3)]

        def _wait_row(r, p, sem):
            for ch, dst in enumerate(_row_slices(r)):
                pltpu.make_async_copy((ob0, ob1, ob2)[ch].at[p], dst,
                                      sem).wait()

        def _do_row(r, p, sem):
            rbase = r * W

            def _tc(tc, _):
                for u in range(8):
                    pbase = rbase + tc * 128 + u * L
                    key = band[pl.ds(pbase, L)]
                    band[pl.ds(pbase, L)] = neg1
                    touched = key >= 0
                    polb = (key & 1) == 1
                    y0 = jnp.where(touched & polb, 0.0, 255.0)
                    y0 = y0.astype(jnp.float32)
                    y1 = jnp.where(touched & (~polb), 0.0, 255.0)
                    y1 = y1.astype(jnp.float32)
                    ob0[p, tc, 0, pl.ds(u * L, L)] = y0
                    ob1[p, tc, 0, pl.ds(u * L, L)] = y1
                    ob2[p, tc, 0, pl.ds(u * L, L)] = y0 + y1
                return 0
            lax.fori_loop(0, 10, _tc, 0)
            for ch, dst in enumerate(_row_slices(r)):
                pltpu.async_copy((ob0, ob1, ob2)[ch].at[p], dst, sem)

        def _rowpair(i, _):
            @pl.when(i > 0)
            def _():
                _wait_row(2 * i - 2, 0, so0)
            _do_row(2 * i, 0, so0)

            @pl.when(i > 0)
            def _():
                _wait_row(2 * i - 1, 1, so1)
            _do_row(2 * i + 1, 1, so1)
            return 0
        lax.fori_loop(0, BAND // 2, _rowpair, 0)

        _wait_row(BAND - 3, 0, so0)
        _do_row(BAND - 1, 0, so0)
        _wait_row(BAND - 2, 1, so1)
        _wait_row(BAND - 1, 0, so0)
        return 0
    lax.fori_loop(0, B // NC, _batch, 0)


@jax.jit
def _e2img(xt, xtail):
    mesh = plsc.VectorSubcoreMesh(
        core_axis_name="c", subcore_axis_name="s", num_cores=NC, num_subcores=NS)
    f = pl.kernel(
        _sc_body,
        out_type=jax.ShapeDtypeStruct(
            (B, 3, H // 8, W // 128, 8, 128), jnp.float32),
        mesh=mesh,
        scratch_types=[
            pltpu.VMEM((BAND * W,), jnp.int32),
            pltpu.VMEM((4, CHN), jnp.float32),
            pltpu.VMEM((4, CHN), jnp.float32),
            pltpu.VMEM((TAILE * 4,), jnp.float32),
            pltpu.VMEM((2, W // 128, 1, 128), jnp.float32),
            pltpu.VMEM((2, W // 128, 1, 128), jnp.float32),
            pltpu.VMEM((2, W // 128, 1, 128), jnp.float32),
            pltpu.SemaphoreType.DMA,
            pltpu.SemaphoreType.DMA,
            pltpu.SemaphoreType.DMA,
            pltpu.SemaphoreType.DMA,
        ],
        compiler_params=pltpu.CompilerParams(needs_layout_passes=False),
    )
    return f(xt, xtail)


def kernel(x):
    xt = x.transpose(0, 2, 1)
    xtail = x[:, N - TAILE:, :].reshape(B, TAILE * 4)
    out = _e2img(xt, xtail)
    return out.transpose(0, 2, 4, 3, 5, 1).reshape(B, H, W, 3)
